# baseline (speedup 1.0000x reference)
import jax
import jax.numpy as jnp
from jax import lax
from jax.experimental import pallas as pl
from jax.experimental.pallas import tpu as pltpu
from jax.experimental.pallas import tpu_sc as plsc

B, L, D = 16384, 50, 64
N = B * L
NC, NS = 2, 16
NW = NC * NS
PER_W = N // NW
CH = 128
S = PER_W // CH
IB = 20
NBLK = S // IB
NJ = B // CH


def _body(it_t, br_t, ma_t, au_t, ii, bi, mi, ai, out,
          ib_ref, r00, r01, r02, r03, r10, r11, r12, r13, tb0, tb1,
          semI, semG0, semG1, semW0, semW1):
    wid = lax.axis_index("s") * NC + lax.axis_index("c")
    row0 = wid * S

    idx_hbms = (ii, bi, mi, ai)
    tabs = (it_t, br_t, ma_t, au_t)
    set0 = (r00, r01, r02, r03)
    set1 = (r10, r11, r12, r13)

    for t in range(4):
        pltpu.sync_copy(idx_hbms[t].at[pl.ds(row0, IB)],
                        ib_ref.at[t, pl.ds(0, IB)])
    for t in range(4):
        pltpu.async_copy(idx_hbms[t].at[pl.ds(row0 + IB, IB)],
                         ib_ref.at[t, pl.ds(IB, IB)], semI)
    for t in range(4):
        pltpu.async_copy(tabs[t].at[ib_ref.at[t, 0]], set0[t], semG0)

    iotav = lax.iota(jnp.int32, 16)
    row_idx = [iotav + g * 16 for g in range(8)]

    def accumulate(rset, tb):
        def dbody(d, c_):
            i = d // 8
            r = d - i * 8
            col_idx = jnp.full((16,), d, jnp.int32)
            for g in range(8):
                v0 = plsc.load_gather(rset[0], [row_idx[g], col_idx])
                v1 = plsc.load_gather(rset[1], [row_idx[g], col_idx])
                v2 = plsc.load_gather(rset[2], [row_idx[g], col_idx])
                v3 = plsc.load_gather(rset[3], [row_idx[g], col_idx])
                tb[i, r, pl.ds(g * 16, 16)] = (v0 + v1) + (v2 + v3)
            return c_
        lax.fori_loop(0, D, dbody, 0)

    def wb_dst(s):
        m = wid * S + s
        l = m // NJ
        j = m - l * NJ
        return out.at[l, :, j]

    def dstep(g, carry):
        s0 = 2 * g
        blk = s0 // IB
        off = s0 - blk * IB
        slot = (blk % 2) * IB
        r_s1 = slot + off + 1
        blk2 = (s0 + 2) // IB
        r_s2 = ((blk2 % 2) * IB) + (s0 + 2 - blk2 * IB)

        for t in range(4):
            pltpu.async_copy(tabs[t].at[ib_ref.at[t, r_s1]], set1[t], semG1)

        @pl.when(jnp.logical_and(off == IB - 2, s0 + 2 < S))
        def _():
            for t in range(4):
                pltpu.make_async_copy(idx_hbms[t].at[pl.ds(row0, IB)],
                                      ib_ref.at[t, pl.ds(0, IB)], semI).wait()

        @pl.when(jnp.logical_and(off == 0,
                                 jnp.logical_and(s0 >= IB,
                                                 s0 < (NBLK - 1) * IB)))
        def _():
            nslot = ((blk + 1) % 2) * IB
            for t in range(4):
                pltpu.async_copy(
                    idx_hbms[t].at[pl.ds(row0 + (blk + 1) * IB, IB)],
                    ib_ref.at[t, pl.ds(nslot, IB)], semI)

        for t in range(4):
            pltpu.make_async_copy(it_t.at[pl.ds(0, CH)], set0[t], semG0).wait()

        @pl.when(g > 0)
        def _():
            pltpu.make_async_copy(tb0, wb_dst(s0), semW0).wait()

        accumulate(set0, tb0)
        pltpu.async_copy(tb0, wb_dst(s0), semW0)

        @pl.when(s0 + 2 < S)
        def _():
            for t in range(4):
                pltpu.async_copy(tabs[t].at[ib_ref.at[t, r_s2]], set0[t],
                                 semG0)

        for t in range(4):
            pltpu.make_async_copy(it_t.at[pl.ds(0, CH)], set1[t], semG1).wait()

        @pl.when(g > 0)
        def _():
            pltpu.make_async_copy(tb1, wb_dst(s0 + 1), semW1).wait()

        accumulate(set1, tb1)
        pltpu.async_copy(tb1, wb_dst(s0 + 1), semW1)
        return carry

    lax.fori_loop(0, S // 2, dstep, 0)

    pltpu.make_async_copy(tb0, wb_dst(S - 2), semW0).wait()
    pltpu.make_async_copy(tb1, wb_dst(S - 1), semW1).wait()


@jax.jit
def _run(item_id, brand, material, author, it_t, br_t, ma_t, au_t):
    mesh = plsc.VectorSubcoreMesh(core_axis_name="c", subcore_axis_name="s")
    k = pl.kernel(
        _body,
        mesh=mesh,
        out_type=jax.ShapeDtypeStruct((L, D // 8, NJ, 8, CH), jnp.float32),
        compiler_params=pltpu.CompilerParams(use_tc_tiling_on_sc=False,
                                             needs_layout_passes=False),
        scratch_types=[
            pltpu.VMEM((4, 2 * IB, CH), jnp.int32),
            pltpu.VMEM((CH, D), jnp.float32),
            pltpu.VMEM((CH, D), jnp.float32),
            pltpu.VMEM((CH, D), jnp.float32),
            pltpu.VMEM((CH, D), jnp.float32),
            pltpu.VMEM((CH, D), jnp.float32),
            pltpu.VMEM((CH, D), jnp.float32),
            pltpu.VMEM((CH, D), jnp.float32),
            pltpu.VMEM((CH, D), jnp.float32),
            pltpu.VMEM((D // 8, 8, CH), jnp.float32),
            pltpu.VMEM((D // 8, 8, CH), jnp.float32),
            pltpu.SemaphoreType.DMA,
            pltpu.SemaphoreType.DMA,
            pltpu.SemaphoreType.DMA,
            pltpu.SemaphoreType.DMA,
            pltpu.SemaphoreType.DMA,
        ],
    )
    r = k(it_t, br_t, ma_t, au_t,
          item_id.T.reshape(N // CH, CH), brand.T.reshape(N // CH, CH),
          material.T.reshape(N // CH, CH), author.T.reshape(N // CH, CH))
    return r.transpose(2, 4, 0, 1, 3).reshape(B, L, D)


def kernel(item_id, brand, material, author, item_table, brand_table,
           material_table, author_table):
    return _run(item_id, brand, material, author,
                item_table, brand_table, material_table, author_table)
